# baseline (speedup 1.0000x reference)
import jax
import jax.numpy as jnp
from jax import lax
from jax.experimental import pallas as pl
from jax.experimental.pallas import tpu as pltpu
from jax.experimental.pallas import tpu_sc as plsc

BATCH = 16384
EMBED_DIM = 64
ROW_PAD = 128
NUM_NODES = 1000000
NUM_RELS = 1000
NUM_CORES = 2
NUM_SUBCORES = 16
NUM_WORKERS = NUM_CORES * NUM_SUBCORES
B_PER_W = BATCH // NUM_WORKERS
CHUNK = 128
CHUNKS_PER_W = B_PER_W // CHUNK
BVEC = 16

N_TILES = NUM_NODES // 128
N_TAIL = NUM_NODES - N_TILES * 128
TILE_ITERS = (N_TILES + NUM_WORKERS - 1) // NUM_WORKERS
SKEW = 133
LANES = 16


def _transpose_body(nt_hbm, tail_hbm, out_hbm,
                    in_b, o_b, t_buf, sem_r0, sem_r1, sem_w0, sem_w1):
    wid = lax.axis_index("s") * NUM_CORES + lax.axis_index("c")
    iota = lax.broadcasted_iota(jnp.int32, (LANES,), 0)
    sem_r = (sem_r0, sem_r1)
    sem_w = (sem_w0, sem_w1)

    @pl.when(wid == 0)
    def _():
        pltpu.sync_copy(tail_hbm, t_buf)
        pltpu.sync_copy(t_buf, out_hbm.at[pl.ds(N_TILES * 128, N_TAIL)])

    def tile_of(j):
        return wid + j * NUM_WORKERS

    @pl.when(tile_of(0) < N_TILES)
    def _():
        pltpu.async_copy(nt_hbm.at[:, pl.ds(tile_of(0) * 128, 128)],
                         in_b.at[0, :, pl.ds(0, 128)], sem_r0)

    def chunk_body(j, carry):
        t = tile_of(j)

        @pl.when(t < N_TILES)
        def _():
            for p2 in range(2):
                @pl.when((j & 1) == p2)
                def _():
                    @pl.when(tile_of(j + 1) < N_TILES)
                    def _():
                        pltpu.async_copy(
                            nt_hbm.at[:, pl.ds(tile_of(j + 1) * 128, 128)],
                            in_b.at[1 - p2, :, pl.ds(0, 128)],
                            sem_r[1 - p2])

                    pltpu.make_async_copy(
                        nt_hbm.at[:, pl.ds(0, 128)],
                        in_b.at[p2, :, pl.ds(0, 128)],
                        sem_r[p2]).wait()

                    @pl.when(j >= 2)
                    def _():
                        pltpu.make_async_copy(
                            o_b.at[p2], out_hbm.at[pl.ds(0, 128)],
                            sem_w[p2]).wait()

                    def row_body(p, carry2):
                        for pp in range(2):
                            row = 2 * p + pp
                            nsp = row + jnp.zeros((LANES,), jnp.int32)
                            for jj in range(4):
                                o_b[p2, row, pl.ds(jj * LANES, LANES)] = (
                                    plsc.load_gather(
                                        in_b.at[p2],
                                        [iota + jj * LANES, nsp]))
                        return carry2

                    lax.fori_loop(0, 64, row_body, 0)

                    pltpu.async_copy(o_b.at[p2],
                                     out_hbm.at[pl.ds(t * 128, 128)],
                                     sem_w[p2])
        return carry

    lax.fori_loop(0, TILE_ITERS, chunk_body, 0)

    n_my = (N_TILES - wid + NUM_WORKERS - 1) // NUM_WORKERS

    @pl.when(n_my >= 1)
    def _():
        for p2 in range(2):
            @pl.when(((n_my - 1) & 1) == p2)
            def _():
                pltpu.make_async_copy(
                    o_b.at[p2], out_hbm.at[pl.ds(0, 128)],
                    sem_w[p2]).wait()

    @pl.when(n_my >= 2)
    def _():
        for p2 in range(2):
            @pl.when((n_my & 1) == p2)
            def _():
                pltpu.make_async_copy(
                    o_b.at[p2], out_hbm.at[pl.ds(0, 128)],
                    sem_w[p2]).wait()


def _gather_body(node_hbm, rel_hbm, head_hbm, relids_hbm, tail_hbm, out_hbm,
                 idx_h, idx_r, idx_t, h_buf, r_buf, t_buf, o_buf,
                 sem_h, sem_r, sem_t):
    wid = lax.axis_index("s") * NUM_CORES + lax.axis_index("c")
    base = wid * B_PER_W
    idx_row = wid * CHUNKS_PER_W

    pltpu.sync_copy(head_hbm.at[pl.ds(idx_row, CHUNKS_PER_W)], idx_h)
    pltpu.sync_copy(relids_hbm.at[pl.ds(idx_row, CHUNKS_PER_W)], idx_r)
    pltpu.sync_copy(tail_hbm.at[pl.ds(idx_row, CHUNKS_PER_W)], idx_t)

    for c in range(CHUNKS_PER_W):
        ch = pltpu.async_copy(node_hbm.at[idx_h.at[c]], h_buf, sem_h)
        cr = pltpu.async_copy(rel_hbm.at[idx_r.at[c]], r_buf, sem_r)
        ct = pltpu.async_copy(node_hbm.at[idx_t.at[c]], t_buf, sem_t)
        ch.wait()
        cr.wait()
        ct.wait()

        def row_body(i, carry):
            for j in range(EMBED_DIM // BVEC):
                sl = pl.ds(j * BVEC, BVEC)
                o_buf[i, sl] = h_buf[i, sl] + r_buf[i, sl] + t_buf[i, sl]
            return carry

        lax.fori_loop(0, CHUNK, row_body, 0)

        pltpu.sync_copy(o_buf, out_hbm.at[pl.ds(base + c * CHUNK, CHUNK)])


@jax.jit
def kernel(head_ids, rel_ids, tail_ids, node_embeddings, rel_weight):
    mesh = plsc.VectorSubcoreMesh(core_axis_name="c", subcore_axis_name="s",
                                  num_cores=NUM_CORES,
                                  num_subcores=NUM_SUBCORES)
    k1 = pl.kernel(
        _transpose_body,
        out_type=jax.ShapeDtypeStruct((NUM_NODES, ROW_PAD), jnp.float32),
        mesh=mesh,
        compiler_params=pltpu.CompilerParams(needs_layout_passes=False),
        scratch_types=[
            pltpu.VMEM((2, EMBED_DIM, SKEW), jnp.float32),
            pltpu.VMEM((2, 128, ROW_PAD), jnp.float32),
            pltpu.VMEM((N_TAIL, ROW_PAD), jnp.float32),
            pltpu.SemaphoreType.DMA,
            pltpu.SemaphoreType.DMA,
            pltpu.SemaphoreType.DMA,
            pltpu.SemaphoreType.DMA,
        ],
    )
    k2 = pl.kernel(
        _gather_body,
        out_type=jax.ShapeDtypeStruct((BATCH, ROW_PAD), jnp.float32),
        mesh=mesh,
        compiler_params=pltpu.CompilerParams(needs_layout_passes=False),
        scratch_types=[
            pltpu.VMEM((CHUNKS_PER_W, CHUNK), jnp.int32),
            pltpu.VMEM((CHUNKS_PER_W, CHUNK), jnp.int32),
            pltpu.VMEM((CHUNKS_PER_W, CHUNK), jnp.int32),
            pltpu.VMEM((CHUNK, ROW_PAD), jnp.float32),
            pltpu.VMEM((CHUNK, ROW_PAD), jnp.float32),
            pltpu.VMEM((CHUNK, ROW_PAD), jnp.float32),
            pltpu.VMEM((CHUNK, ROW_PAD), jnp.float32),
            pltpu.SemaphoreType.DMA,
            pltpu.SemaphoreType.DMA,
            pltpu.SemaphoreType.DMA,
        ],
    )
    tail_pad = jnp.pad(node_embeddings[N_TILES * 128:],
                       ((0, 0), (0, ROW_PAD - EMBED_DIM)))
    node_bf = k1(node_embeddings.T, tail_pad)
    rel_bf = jnp.pad(rel_weight,
                     ((0, 0), (0, ROW_PAD - EMBED_DIM)))
    nrows = NUM_WORKERS * CHUNKS_PER_W
    head2d = head_ids.reshape(nrows, CHUNK)
    rel2d = rel_ids.reshape(nrows, CHUNK)
    tail2d = tail_ids.reshape(nrows, CHUNK)
    out_pad = k2(node_bf, rel_bf, head2d, rel2d, tail2d)
    return out_pad[:, :EMBED_DIM]


# speedup vs baseline: 4.0668x; 4.0668x over previous
import jax
import jax.numpy as jnp
from jax import lax
from jax.experimental import pallas as pl
from jax.experimental.pallas import tpu as pltpu
from jax.experimental.pallas import tpu_sc as plsc

BATCH = 16384
EMBED_DIM = 64
ROW_PAD = 128
NUM_CORES = 2
NUM_SUBCORES = 16
NUM_WORKERS = NUM_CORES * NUM_SUBCORES
B_PER_W = BATCH // NUM_WORKERS
HALF = B_PER_W // 4
LANES = 16
VECS_PER_ROW = EMBED_DIM // LANES


def _body(node_hbm, rel_hbm, head_hbm, relids_hbm, tail_hbm, out_hbm,
          vidx_h, vidx_r, vidx_t,
          h_buf, r_buf, t_buf, o_buf,
          sem_h, sem_r, sem_t):
    wid = lax.axis_index("s") * NUM_CORES + lax.axis_index("c")
    base = wid * B_PER_W

    pltpu.sync_copy(head_hbm.at[pl.ds(base, B_PER_W)],
                    vidx_h.at[pl.ds(0, B_PER_W)])
    pltpu.sync_copy(relids_hbm.at[pl.ds(base, B_PER_W)],
                    vidx_r.at[pl.ds(0, B_PER_W)])
    pltpu.sync_copy(tail_hbm.at[pl.ds(base, B_PER_W)],
                    vidx_t.at[pl.ds(0, B_PER_W)])

    for half in range(4):
        off = half * HALF

        def issue_body(i, carry):
            hid = vidx_h[pl.ds(off + i, LANES)][0]
            rid = vidx_r[pl.ds(off + i, LANES)][0]
            tid = vidx_t[pl.ds(off + i, LANES)][0]
            pltpu.async_copy(node_hbm.at[pl.ds(hid, 1)],
                             h_buf.at[pl.ds(i, 1)], sem_h)
            pltpu.async_copy(rel_hbm.at[pl.ds(rid, 1)],
                             r_buf.at[pl.ds(i, 1)], sem_r)
            pltpu.async_copy(node_hbm.at[pl.ds(tid, 1)],
                             t_buf.at[pl.ds(i, 1)], sem_t)
            return carry

        lax.fori_loop(0, HALF, issue_body, 0)

        pltpu.make_async_copy(node_hbm.at[pl.ds(0, HALF)], h_buf,
                              sem_h).wait()
        pltpu.make_async_copy(rel_hbm.at[pl.ds(0, HALF)], r_buf,
                              sem_r).wait()
        pltpu.make_async_copy(node_hbm.at[pl.ds(0, HALF)], t_buf,
                              sem_t).wait()

        def row_body(i, carry):
            for j in range(VECS_PER_ROW):
                sl = pl.ds(j * LANES, LANES)
                o_buf[i, sl] = h_buf[i, sl] + r_buf[i, sl] + t_buf[i, sl]
            return carry

        lax.fori_loop(0, HALF, row_body, 0)

        pltpu.sync_copy(o_buf, out_hbm.at[pl.ds(base + off, HALF)])


@jax.jit
def kernel(head_ids, rel_ids, tail_ids, node_embeddings, rel_weight):
    mesh = plsc.VectorSubcoreMesh(core_axis_name="c", subcore_axis_name="s",
                                  num_cores=NUM_CORES,
                                  num_subcores=NUM_SUBCORES)
    k = pl.kernel(
        _body,
        out_type=jax.ShapeDtypeStruct((BATCH, ROW_PAD), jnp.float32),
        mesh=mesh,
        compiler_params=pltpu.CompilerParams(needs_layout_passes=False),
        scratch_types=[
            pltpu.VMEM((B_PER_W + LANES,), jnp.int32),
            pltpu.VMEM((B_PER_W + LANES,), jnp.int32),
            pltpu.VMEM((B_PER_W + LANES,), jnp.int32),
            pltpu.VMEM((HALF, EMBED_DIM), jnp.float32),
            pltpu.VMEM((HALF, EMBED_DIM), jnp.float32),
            pltpu.VMEM((HALF, EMBED_DIM), jnp.float32),
            pltpu.VMEM((HALF, ROW_PAD), jnp.float32),
            pltpu.SemaphoreType.DMA,
            pltpu.SemaphoreType.DMA,
            pltpu.SemaphoreType.DMA,
        ],
    )
    out_pad = k(node_embeddings, rel_weight, head_ids, rel_ids, tail_ids)
    return out_pad[:, :EMBED_DIM]
